# Initial kernel scaffold; baseline (speedup 1.0000x reference)
import jax
import jax.numpy as jnp
from jax.experimental import pallas as pl


def kernel(x, edge_index, batch, W1, b1, W2, b2, W3, b3, Wlin, blin, Wout, bout):
    raise NotImplementedError("write your pallas kernel here")


# speedup vs baseline: 12.4498x; 12.4498x over previous
import functools

import jax
import jax.numpy as jnp
from jax import lax
from jax.experimental import pallas as pl
from jax.experimental.pallas import tpu as pltpu
from jax.experimental.pallas import tpu_sc as plsc

N = 10000
E = 320000
D = 128
H = 128
C = 32
G = 64

NP = 10240
NC, NS = 2, 16
NW = NC * NS
EPW = E // NW
EC = 80
NCHUNK = EPW // EC
RPT = NP // NS
RBLK = 80
DPT = NP // NS

BR = 512
NBLK = NP // BR

@functools.lru_cache(maxsize=None)
def _mesh():
    return plsc.VectorSubcoreMesh(
        core_axis_name="c", subcore_axis_name="s", num_cores=NC, num_subcores=NS)


def _deg_body(dst_hbm, out_hbm, deg_sh, idx_all, ones_v, buf_v):
    c = lax.axis_index("c")
    s = lax.axis_index("s")
    wid = c * NS + s
    for i in range(EC // 16):
        ones_v[pl.ds(16 * i, 16)] = jnp.ones((16,), jnp.float32)
    for i in range(DPT // 16):
        buf_v[pl.ds(16 * i, 16)] = jnp.zeros((16,), jnp.float32)
    pltpu.sync_copy(buf_v, deg_sh.at[pl.ds(s * DPT, DPT)])
    pltpu.sync_copy(dst_hbm.at[wid], idx_all)
    plsc.subcore_barrier()

    def step(i, _):
        pltpu.sync_copy(ones_v, deg_sh.at[idx_all.at[i]], add=True)
        return ()

    lax.fori_loop(0, NCHUNK, step, ())
    plsc.subcore_barrier()
    pltpu.sync_copy(deg_sh.at[pl.ds(s * DPT, DPT)], buf_v)
    pltpu.sync_copy(buf_v, out_hbm.at[c, pl.ds(s * DPT, DPT)])


@functools.lru_cache(maxsize=None)
def _deg_call():
    return pl.kernel(
        _deg_body,
        out_type=jax.ShapeDtypeStruct((NC, NP), jnp.float32),
        mesh=_mesh(),
        scratch_types=[
            pltpu.VMEM_SHARED((NP,), jnp.float32),
            pltpu.VMEM((NCHUNK, EC), jnp.int32),
            pltpu.VMEM((EC,), jnp.float32),
            pltpu.VMEM((DPT,), jnp.float32),
        ],
    )


def _edge_body(y_hbm, seed_hbm, src_hbm, dst_hbm, out_hbm,
               agg_sh, sidx, didx, rows_v, sem):
    c = lax.axis_index("c")
    s = lax.axis_index("s")
    wid = c * NS + s
    row0 = s * RPT
    pltpu.sync_copy(src_hbm.at[wid], sidx)
    pltpu.sync_copy(dst_hbm.at[wid], didx)

    def seed(i, _):
        r = row0 + i * RBLK

        @pl.when(c == 0)
        def _():
            pltpu.sync_copy(y_hbm.at[pl.ds(r, RBLK)], rows_v)

        @pl.when(c != 0)
        def _():
            pltpu.sync_copy(seed_hbm.at[pl.ds(r, RBLK)], rows_v)

        pltpu.sync_copy(rows_v, agg_sh.at[pl.ds(r, RBLK)])
        return ()

    lax.fori_loop(0, RPT // RBLK, seed, ())
    plsc.subcore_barrier()

    def step(i, _):
        pltpu.async_copy(y_hbm.at[sidx.at[i]], rows_v, sem).wait()
        pltpu.sync_copy(rows_v, agg_sh.at[didx.at[i]], add=True)
        return ()

    lax.fori_loop(0, NCHUNK, step, ())
    plsc.subcore_barrier()

    def cpout(i, _):
        r = row0 + i * RBLK
        pltpu.sync_copy(agg_sh.at[pl.ds(r, RBLK)], rows_v)
        pltpu.sync_copy(rows_v, out_hbm.at[c, pl.ds(r, RBLK)])
        return ()

    lax.fori_loop(0, RPT // RBLK, cpout, ())


@functools.lru_cache(maxsize=None)
def _edge_call():
    return pl.kernel(
        _edge_body,
        out_type=jax.ShapeDtypeStruct((NC, NP, D), jnp.float32),
        mesh=_mesh(),
        scratch_types=[
            pltpu.VMEM_SHARED((NP, D), jnp.float32),
            pltpu.VMEM((NCHUNK, EC), jnp.int32),
            pltpu.VMEM((NCHUNK, EC), jnp.int32),
            pltpu.VMEM((EC, D), jnp.float32),
            pltpu.SemaphoreType.DMA,
        ],
    )


def _y1_body(x_ref, d0_ref, d1_ref, w_ref, y_ref):
    dinv = lax.rsqrt(d0_ref[...] + d1_ref[...] + 1.0)
    xw = jnp.dot(x_ref[...], w_ref[...], preferred_element_type=jnp.float32)
    y_ref[...] = xw * dinv


_y1_call = pl.pallas_call(
    _y1_body,
    grid=(NBLK,),
    in_specs=[
        pl.BlockSpec((BR, D), lambda i: (i, 0)),
        pl.BlockSpec((BR, 1), lambda i: (i, 0)),
        pl.BlockSpec((BR, 1), lambda i: (i, 0)),
        pl.BlockSpec((D, H), lambda i: (0, 0)),
    ],
    out_specs=pl.BlockSpec((BR, H), lambda i: (i, 0)),
    out_shape=jax.ShapeDtypeStruct((NP, H), jnp.float32),
)


def _layer_body(p0_ref, p1_ref, d0_ref, d1_ref, b_ref, w_ref, y_ref):
    dinv = lax.rsqrt(d0_ref[...] + d1_ref[...] + 1.0)
    t = (p0_ref[...] + p1_ref[...]) * dinv + b_ref[...]
    h = jnp.where(t > 0, t, 0.01 * t)
    y_ref[...] = jnp.dot(h, w_ref[...], preferred_element_type=jnp.float32) * dinv


_layer_call = pl.pallas_call(
    _layer_body,
    grid=(NBLK,),
    in_specs=[
        pl.BlockSpec((BR, H), lambda i: (i, 0)),
        pl.BlockSpec((BR, H), lambda i: (i, 0)),
        pl.BlockSpec((BR, 1), lambda i: (i, 0)),
        pl.BlockSpec((BR, 1), lambda i: (i, 0)),
        pl.BlockSpec((1, H), lambda i: (0, 0)),
        pl.BlockSpec((H, H), lambda i: (0, 0)),
    ],
    out_specs=pl.BlockSpec((BR, H), lambda i: (i, 0)),
    out_shape=jax.ShapeDtypeStruct((NP, H), jnp.float32),
)


def _final_body(p0_ref, p1_ref, d0_ref, d1_ref, b3_ref, batch_ref,
                wl_ref, bl_ref, wo_ref, bo_ref,
                logits_ref, probs_ref, feats_ref, embeds_ref, hout_ref,
                acc_ref, cnt_ref):
    i = pl.program_id(0)

    @pl.when(i == 0)
    def _():
        acc_ref[...] = jnp.zeros_like(acc_ref)
        cnt_ref[...] = jnp.zeros_like(cnt_ref)

    dinv = lax.rsqrt(d0_ref[...] + d1_ref[...] + 1.0)
    h3 = (p0_ref[...] + p1_ref[...]) * dinv + b3_ref[...]
    ids = batch_ref[0, 0, :]
    seg = lax.broadcasted_iota(jnp.int32, (G, BR), 0)
    onehot = jnp.where(seg == ids[None, :], 1.0, 0.0)
    acc_ref[...] += jnp.dot(onehot, h3, preferred_element_type=jnp.float32,
                            precision=lax.Precision.HIGHEST)
    cnt_ref[...] += jnp.sum(onehot, axis=1, keepdims=True)

    @pl.when(i == NBLK - 1)
    def _():
        summ = acc_ref[...]
        mean = summ / jnp.maximum(cnt_ref[...], 1.0)
        feats = jnp.concatenate([summ, mean], axis=1)
        embeds = jnp.dot(feats, wl_ref[...],
                         preferred_element_type=jnp.float32) + bl_ref[...]
        hout = jnp.maximum(embeds, 0.0)
        logits = jnp.dot(hout, wo_ref[...],
                         preferred_element_type=jnp.float32) + bo_ref[...]
        m = jnp.max(logits, axis=1, keepdims=True)
        ex = jnp.exp(logits - m)
        feats_ref[...] = feats
        embeds_ref[...] = embeds
        hout_ref[...] = hout
        logits_ref[...] = logits
        probs_ref[...] = ex / jnp.sum(ex, axis=1, keepdims=True)


_final_call = pl.pallas_call(
    _final_body,
    grid=(NBLK,),
    in_specs=[
        pl.BlockSpec((BR, H), lambda i: (i, 0)),
        pl.BlockSpec((BR, H), lambda i: (i, 0)),
        pl.BlockSpec((BR, 1), lambda i: (i, 0)),
        pl.BlockSpec((BR, 1), lambda i: (i, 0)),
        pl.BlockSpec((1, H), lambda i: (0, 0)),
        pl.BlockSpec((1, 1, BR), lambda i: (i, 0, 0)),
        pl.BlockSpec((2 * H, H), lambda i: (0, 0)),
        pl.BlockSpec((1, H), lambda i: (0, 0)),
        pl.BlockSpec((H, C), lambda i: (0, 0)),
        pl.BlockSpec((1, C), lambda i: (0, 0)),
    ],
    out_specs=[
        pl.BlockSpec((G, C), lambda i: (0, 0)),
        pl.BlockSpec((G, C), lambda i: (0, 0)),
        pl.BlockSpec((G, 2 * H), lambda i: (0, 0)),
        pl.BlockSpec((G, H), lambda i: (0, 0)),
        pl.BlockSpec((G, H), lambda i: (0, 0)),
    ],
    out_shape=[
        jax.ShapeDtypeStruct((G, C), jnp.float32),
        jax.ShapeDtypeStruct((G, C), jnp.float32),
        jax.ShapeDtypeStruct((G, 2 * H), jnp.float32),
        jax.ShapeDtypeStruct((G, H), jnp.float32),
        jax.ShapeDtypeStruct((G, H), jnp.float32),
    ],
    scratch_shapes=[
        pltpu.VMEM((G, H), jnp.float32),
        pltpu.VMEM((G, 1), jnp.float32),
    ],
)


def kernel(x, edge_index, batch, W1, b1, W2, b2, W3, b3, Wlin, blin, Wout, bout):
    src3 = edge_index[0].astype(jnp.int32).reshape(NW, NCHUNK, EC)
    dst3 = edge_index[1].astype(jnp.int32).reshape(NW, NCHUNK, EC)
    batch3 = jnp.concatenate(
        [batch.astype(jnp.int32), jnp.full((NP - N,), G, jnp.int32)]
    ).reshape(NBLK, 1, BR)
    xp = jnp.pad(x, ((0, NP - N), (0, 0)))
    zeros_nd = jnp.zeros((NP, D), jnp.float32)

    deg_parts = _deg_call()(dst3)
    d0 = deg_parts[0].reshape(NP, 1)
    d1 = deg_parts[1].reshape(NP, 1)

    y1 = _y1_call(xp, d0, d1, W1)
    p1 = _edge_call()(y1, zeros_nd, src3, dst3)
    y2 = _layer_call(p1[0], p1[1], d0, d1, b1.reshape(1, H), W2)
    p2 = _edge_call()(y2, zeros_nd, src3, dst3)
    y3 = _layer_call(p2[0], p2[1], d0, d1, b2.reshape(1, H), W3)
    p3 = _edge_call()(y3, zeros_nd, src3, dst3)

    logits, probs, feats, embeds, hout = _final_call(
        p3[0], p3[1], d0, d1, b3.reshape(1, H), batch3,
        Wlin, blin.reshape(1, H), Wout, bout.reshape(1, C))
    return (logits, probs, feats, embeds, hout)


# speedup vs baseline: 17.6247x; 1.4157x over previous
import functools

import jax
import jax.numpy as jnp
from jax import lax
from jax.experimental import pallas as pl
from jax.experimental.pallas import tpu as pltpu
from jax.experimental.pallas import tpu_sc as plsc

N = 10000
E = 320000
D = 128
H = 128
C = 32
G = 64

NP = 10240
NC, NS = 2, 16
NW = NC * NS
EPW = E // NW
EC = 80
NCHUNK = EPW // EC
RPT = NP // NS
RBLK = 80
DPT = NP // NS
DSTG = 25

BR = 512
NBLK = NP // BR

@functools.lru_cache(maxsize=None)
def _mesh():
    return plsc.VectorSubcoreMesh(
        core_axis_name="c", subcore_axis_name="s", num_cores=NC, num_subcores=NS)


def _deg_body(dst_hbm, out_hbm, deg_sh, idx_st, ones_v, buf_v):
    c = lax.axis_index("c")
    s = lax.axis_index("s")
    wid = c * NS + s
    for i in range(EC // 16):
        ones_v[pl.ds(16 * i, 16)] = jnp.ones((16,), jnp.float32)
    for i in range(DPT // 16):
        buf_v[pl.ds(16 * i, 16)] = jnp.zeros((16,), jnp.float32)
    pltpu.sync_copy(buf_v, deg_sh.at[pl.ds(s * DPT, DPT)])
    plsc.subcore_barrier()

    def stage(st, _):
        pltpu.sync_copy(dst_hbm.at[wid, st], idx_st)

        def step(i, _):
            pltpu.sync_copy(ones_v, deg_sh.at[idx_st.at[i]], add=True)
            return ()

        lax.fori_loop(0, DSTG, step, ())
        return ()

    lax.fori_loop(0, NCHUNK // DSTG, stage, ())
    plsc.subcore_barrier()
    pltpu.sync_copy(deg_sh.at[pl.ds(s * DPT, DPT)], buf_v)
    pltpu.sync_copy(buf_v, out_hbm.at[c, pl.ds(s * DPT, DPT)])


@functools.lru_cache(maxsize=None)
def _deg_call():
    return pl.kernel(
        _deg_body,
        out_type=jax.ShapeDtypeStruct((NC, NP), jnp.float32),
        mesh=_mesh(),
        scratch_types=[
            pltpu.VMEM_SHARED((NP,), jnp.float32),
            pltpu.VMEM((DSTG, EC), jnp.int32),
            pltpu.VMEM((EC,), jnp.float32),
            pltpu.VMEM((DPT,), jnp.float32),
        ],
    )


def _edge_body(y_hbm, seed_hbm, src_hbm, dst_hbm, out_hbm,
               agg_sh, sidx, didx, rows_v, rows2_v, sem, sem2):
    c = lax.axis_index("c")
    s = lax.axis_index("s")
    wid = c * NS + s
    row0 = s * RPT

    def seed(i, _):
        r = row0 + i * RBLK

        @pl.when(c == 0)
        def _():
            pltpu.sync_copy(y_hbm.at[pl.ds(r, RBLK)], rows_v)

        @pl.when(c != 0)
        def _():
            pltpu.sync_copy(seed_hbm.at[pl.ds(r, RBLK)], rows_v)

        pltpu.sync_copy(rows_v, agg_sh.at[pl.ds(r, RBLK)])
        return ()

    lax.fori_loop(0, RPT // RBLK, seed, ())
    plsc.subcore_barrier()

    def stage(st, _):
        pltpu.sync_copy(src_hbm.at[wid, st], sidx)
        pltpu.sync_copy(dst_hbm.at[wid, st], didx)
        pltpu.async_copy(y_hbm.at[sidx.at[0]], rows_v, sem)

        def pair(j, _):
            ca, cb = 2 * j, 2 * j + 1
            pltpu.async_copy(y_hbm.at[sidx.at[cb]], rows2_v, sem2)
            pltpu.make_async_copy(y_hbm.at[sidx.at[ca]], rows_v, sem).wait()
            pltpu.sync_copy(rows_v, agg_sh.at[didx.at[ca]], add=True)
            pltpu.async_copy(y_hbm.at[sidx.at[ca + 2]], rows_v, sem)
            pltpu.make_async_copy(y_hbm.at[sidx.at[cb]], rows2_v, sem2).wait()
            pltpu.sync_copy(rows2_v, agg_sh.at[didx.at[cb]], add=True)
            return ()

        lax.fori_loop(0, (DSTG - 1) // 2, pair, ())
        last = DSTG - 1
        pltpu.make_async_copy(y_hbm.at[sidx.at[last]], rows_v, sem).wait()
        pltpu.sync_copy(rows_v, agg_sh.at[didx.at[last]], add=True)
        return ()

    lax.fori_loop(0, NCHUNK // DSTG, stage, ())
    plsc.subcore_barrier()

    def cpout(i, _):
        r = row0 + i * RBLK
        pltpu.sync_copy(agg_sh.at[pl.ds(r, RBLK)], rows_v)
        pltpu.sync_copy(rows_v, out_hbm.at[c, pl.ds(r, RBLK)])
        return ()

    lax.fori_loop(0, RPT // RBLK, cpout, ())


@functools.lru_cache(maxsize=None)
def _edge_call():
    return pl.kernel(
        _edge_body,
        out_type=jax.ShapeDtypeStruct((NC, NP, D), jnp.float32),
        mesh=_mesh(),
        scratch_types=[
            pltpu.VMEM_SHARED((NP, D), jnp.float32),
            pltpu.VMEM((DSTG, EC), jnp.int32),
            pltpu.VMEM((DSTG, EC), jnp.int32),
            pltpu.VMEM((EC, D), jnp.float32),
            pltpu.VMEM((EC, D), jnp.float32),
            pltpu.SemaphoreType.DMA,
            pltpu.SemaphoreType.DMA,
        ],
    )


def _y1_body(x_ref, d0_ref, d1_ref, w_ref, y_ref):
    dinv = lax.rsqrt(d0_ref[...] + d1_ref[...] + 1.0)
    xw = jnp.dot(x_ref[...], w_ref[...], preferred_element_type=jnp.float32)
    y_ref[...] = xw * dinv


_y1_call = pl.pallas_call(
    _y1_body,
    grid=(NBLK,),
    in_specs=[
        pl.BlockSpec((BR, D), lambda i: (i, 0)),
        pl.BlockSpec((BR, 1), lambda i: (i, 0)),
        pl.BlockSpec((BR, 1), lambda i: (i, 0)),
        pl.BlockSpec((D, H), lambda i: (0, 0)),
    ],
    out_specs=pl.BlockSpec((BR, H), lambda i: (i, 0)),
    out_shape=jax.ShapeDtypeStruct((NP, H), jnp.float32),
)


def _layer_body(p0_ref, p1_ref, d0_ref, d1_ref, b_ref, w_ref, y_ref):
    dinv = lax.rsqrt(d0_ref[...] + d1_ref[...] + 1.0)
    t = (p0_ref[...] + p1_ref[...]) * dinv + b_ref[...]
    h = jnp.where(t > 0, t, 0.01 * t)
    y_ref[...] = jnp.dot(h, w_ref[...], preferred_element_type=jnp.float32) * dinv


_layer_call = pl.pallas_call(
    _layer_body,
    grid=(NBLK,),
    in_specs=[
        pl.BlockSpec((BR, H), lambda i: (i, 0)),
        pl.BlockSpec((BR, H), lambda i: (i, 0)),
        pl.BlockSpec((BR, 1), lambda i: (i, 0)),
        pl.BlockSpec((BR, 1), lambda i: (i, 0)),
        pl.BlockSpec((1, H), lambda i: (0, 0)),
        pl.BlockSpec((H, H), lambda i: (0, 0)),
    ],
    out_specs=pl.BlockSpec((BR, H), lambda i: (i, 0)),
    out_shape=jax.ShapeDtypeStruct((NP, H), jnp.float32),
)


def _final_body(p0_ref, p1_ref, d0_ref, d1_ref, b3_ref, batch_ref,
                wl_ref, bl_ref, wo_ref, bo_ref,
                logits_ref, probs_ref, feats_ref, embeds_ref, hout_ref,
                acc_ref, cnt_ref):
    i = pl.program_id(0)

    @pl.when(i == 0)
    def _():
        acc_ref[...] = jnp.zeros_like(acc_ref)
        cnt_ref[...] = jnp.zeros_like(cnt_ref)

    dinv = lax.rsqrt(d0_ref[...] + d1_ref[...] + 1.0)
    h3 = (p0_ref[...] + p1_ref[...]) * dinv + b3_ref[...]
    ids = batch_ref[0, 0, :]
    seg = lax.broadcasted_iota(jnp.int32, (G, BR), 0)
    onehot = jnp.where(seg == ids[None, :], 1.0, 0.0)
    acc_ref[...] += jnp.dot(onehot, h3, preferred_element_type=jnp.float32,
                            precision=lax.Precision.HIGHEST)
    cnt_ref[...] += jnp.sum(onehot, axis=1, keepdims=True)

    @pl.when(i == NBLK - 1)
    def _():
        summ = acc_ref[...]
        mean = summ / jnp.maximum(cnt_ref[...], 1.0)
        feats = jnp.concatenate([summ, mean], axis=1)
        embeds = jnp.dot(feats, wl_ref[...],
                         preferred_element_type=jnp.float32) + bl_ref[...]
        hout = jnp.maximum(embeds, 0.0)
        logits = jnp.dot(hout, wo_ref[...],
                         preferred_element_type=jnp.float32) + bo_ref[...]
        m = jnp.max(logits, axis=1, keepdims=True)
        ex = jnp.exp(logits - m)
        feats_ref[...] = feats
        embeds_ref[...] = embeds
        hout_ref[...] = hout
        logits_ref[...] = logits
        probs_ref[...] = ex / jnp.sum(ex, axis=1, keepdims=True)


_final_call = pl.pallas_call(
    _final_body,
    grid=(NBLK,),
    in_specs=[
        pl.BlockSpec((BR, H), lambda i: (i, 0)),
        pl.BlockSpec((BR, H), lambda i: (i, 0)),
        pl.BlockSpec((BR, 1), lambda i: (i, 0)),
        pl.BlockSpec((BR, 1), lambda i: (i, 0)),
        pl.BlockSpec((1, H), lambda i: (0, 0)),
        pl.BlockSpec((1, 1, BR), lambda i: (i, 0, 0)),
        pl.BlockSpec((2 * H, H), lambda i: (0, 0)),
        pl.BlockSpec((1, H), lambda i: (0, 0)),
        pl.BlockSpec((H, C), lambda i: (0, 0)),
        pl.BlockSpec((1, C), lambda i: (0, 0)),
    ],
    out_specs=[
        pl.BlockSpec((G, C), lambda i: (0, 0)),
        pl.BlockSpec((G, C), lambda i: (0, 0)),
        pl.BlockSpec((G, 2 * H), lambda i: (0, 0)),
        pl.BlockSpec((G, H), lambda i: (0, 0)),
        pl.BlockSpec((G, H), lambda i: (0, 0)),
    ],
    out_shape=[
        jax.ShapeDtypeStruct((G, C), jnp.float32),
        jax.ShapeDtypeStruct((G, C), jnp.float32),
        jax.ShapeDtypeStruct((G, 2 * H), jnp.float32),
        jax.ShapeDtypeStruct((G, H), jnp.float32),
        jax.ShapeDtypeStruct((G, H), jnp.float32),
    ],
    scratch_shapes=[
        pltpu.VMEM((G, H), jnp.float32),
        pltpu.VMEM((G, 1), jnp.float32),
    ],
)


def kernel(x, edge_index, batch, W1, b1, W2, b2, W3, b3, Wlin, blin, Wout, bout):
    src4 = edge_index[0].astype(jnp.int32).reshape(NW, NCHUNK // DSTG, DSTG, EC)
    dst4 = edge_index[1].astype(jnp.int32).reshape(NW, NCHUNK // DSTG, DSTG, EC)
    batch3 = jnp.concatenate(
        [batch.astype(jnp.int32), jnp.full((NP - N,), G, jnp.int32)]
    ).reshape(NBLK, 1, BR)
    xp = jnp.pad(x, ((0, NP - N), (0, 0)))
    zeros_nd = jnp.zeros((NP, D), jnp.float32)

    deg_parts = _deg_call()(dst4)
    d0 = deg_parts[0].reshape(NP, 1)
    d1 = deg_parts[1].reshape(NP, 1)

    y1 = _y1_call(xp, d0, d1, W1)
    p1 = _edge_call()(y1, zeros_nd, src4, dst4)
    y2 = _layer_call(p1[0], p1[1], d0, d1, b1.reshape(1, H), W2)
    p2 = _edge_call()(y2, zeros_nd, src4, dst4)
    y3 = _layer_call(p2[0], p2[1], d0, d1, b2.reshape(1, H), W3)
    p3 = _edge_call()(y3, zeros_nd, src4, dst4)

    logits, probs, feats, embeds, hout = _final_call(
        p3[0], p3[1], d0, d1, b3.reshape(1, H), batch3,
        Wlin, blin.reshape(1, H), Wout, bout.reshape(1, C))
    return (logits, probs, feats, embeds, hout)


# speedup vs baseline: 20.2163x; 1.1470x over previous
import functools

import jax
import jax.numpy as jnp
from jax import lax
from jax.experimental import pallas as pl
from jax.experimental.pallas import tpu as pltpu
from jax.experimental.pallas import tpu_sc as plsc

N = 10000
E = 320000
D = 128
H = 128
C = 32
G = 64

NP = 10240
NC, NS = 2, 16
NW = NC * NS
EPW = 10240
EP = NW * EPW
EC = 128
NCHUNK = EPW // EC
RPT = NP // NS
RBLK = 128
DPT = NP // NS
DSTG = 16

BR = 512
NBLK = NP // BR

@functools.lru_cache(maxsize=None)
def _mesh():
    return plsc.VectorSubcoreMesh(
        core_axis_name="c", subcore_axis_name="s", num_cores=NC, num_subcores=NS)


def _deg_body(dst_hbm, out_hbm, deg_sh, idx_st, ones_v, buf_v):
    c = lax.axis_index("c")
    s = lax.axis_index("s")
    wid = c * NS + s
    for i in range(EC // 16):
        ones_v[pl.ds(16 * i, 16)] = jnp.ones((16,), jnp.float32)
    for i in range(DPT // 16):
        buf_v[pl.ds(16 * i, 16)] = jnp.zeros((16,), jnp.float32)
    pltpu.sync_copy(buf_v, deg_sh.at[pl.ds(s * DPT, DPT)])
    plsc.subcore_barrier()

    def stage(st, _):
        pltpu.sync_copy(dst_hbm.at[wid, st], idx_st)

        def step(i, _):
            pltpu.sync_copy(ones_v, deg_sh.at[idx_st.at[i]], add=True)
            return ()

        lax.fori_loop(0, DSTG, step, ())
        return ()

    lax.fori_loop(0, NCHUNK // DSTG, stage, ())
    plsc.subcore_barrier()
    pltpu.sync_copy(deg_sh.at[pl.ds(s * DPT, DPT)], buf_v)
    pltpu.sync_copy(buf_v, out_hbm.at[c, pl.ds(s * DPT, DPT)])


@functools.lru_cache(maxsize=None)
def _deg_call():
    return pl.kernel(
        _deg_body,
        out_type=jax.ShapeDtypeStruct((NC, NP), jnp.float32),
        mesh=_mesh(),
        scratch_types=[
            pltpu.VMEM_SHARED((NP,), jnp.float32),
            pltpu.VMEM((DSTG, EC), jnp.int32),
            pltpu.VMEM((EC,), jnp.float32),
            pltpu.VMEM((DPT,), jnp.float32),
        ],
    )


def _edge_body(y_hbm, seed_hbm, src_hbm, dst_hbm, out0_hbm, out1_hbm,
               agg_sh, sidx, didx, rows_v, rows2_v, sem, sem2):
    c = lax.axis_index("c")
    s = lax.axis_index("s")
    wid = c * NS + s
    row0 = s * RPT

    def seed(i, _):
        r = row0 + i * RBLK

        @pl.when(c == 0)
        def _():
            pltpu.sync_copy(y_hbm.at[pl.ds(r, RBLK)], rows_v)

        @pl.when(c != 0)
        def _():
            pltpu.sync_copy(seed_hbm.at[pl.ds(r, RBLK)], rows_v)

        pltpu.sync_copy(rows_v, agg_sh.at[pl.ds(r, RBLK)])
        return ()

    lax.fori_loop(0, RPT // RBLK, seed, ())
    plsc.subcore_barrier()

    def stage(st, _):
        pltpu.sync_copy(src_hbm.at[wid, st], sidx)
        pltpu.sync_copy(dst_hbm.at[wid, st], didx)
        pltpu.async_copy(y_hbm.at[sidx.at[0]], rows_v, sem)

        def pair(j, _):
            ca, cb = 2 * j, 2 * j + 1
            pltpu.async_copy(y_hbm.at[sidx.at[cb]], rows2_v, sem2)
            pltpu.make_async_copy(y_hbm.at[sidx.at[ca]], rows_v, sem).wait()
            pltpu.sync_copy(rows_v, agg_sh.at[didx.at[ca]], add=True)
            pltpu.async_copy(y_hbm.at[sidx.at[ca + 2]], rows_v, sem)
            pltpu.make_async_copy(y_hbm.at[sidx.at[cb]], rows2_v, sem2).wait()
            pltpu.sync_copy(rows2_v, agg_sh.at[didx.at[cb]], add=True)
            return ()

        lax.fori_loop(0, (DSTG - 1) // 2, pair, ())
        last = DSTG - 1
        pltpu.make_async_copy(y_hbm.at[sidx.at[last]], rows_v, sem).wait()
        pltpu.sync_copy(rows_v, agg_sh.at[didx.at[last]], add=True)
        return ()

    lax.fori_loop(0, NCHUNK // DSTG, stage, ())
    plsc.subcore_barrier()

    def cpout(i, _):
        r = row0 + i * RBLK
        pltpu.sync_copy(agg_sh.at[pl.ds(r, RBLK)], rows_v)

        @pl.when(c == 0)
        def _():
            pltpu.sync_copy(rows_v, out0_hbm.at[pl.ds(r, RBLK)])

        @pl.when(c != 0)
        def _():
            pltpu.sync_copy(rows_v, out1_hbm.at[pl.ds(r, RBLK)])

        return ()

    lax.fori_loop(0, RPT // RBLK, cpout, ())


@functools.lru_cache(maxsize=None)
def _edge_call():
    return pl.kernel(
        _edge_body,
        out_type=(jax.ShapeDtypeStruct((NP, D), jnp.float32),
                  jax.ShapeDtypeStruct((NP, D), jnp.float32)),
        mesh=_mesh(),
        scratch_types=[
            pltpu.VMEM_SHARED((NP, D), jnp.float32),
            pltpu.VMEM((DSTG, EC), jnp.int32),
            pltpu.VMEM((DSTG, EC), jnp.int32),
            pltpu.VMEM((EC, D), jnp.float32),
            pltpu.VMEM((EC, D), jnp.float32),
            pltpu.SemaphoreType.DMA,
            pltpu.SemaphoreType.DMA,
        ],
    )


def _y1_body(x_ref, d0_ref, d1_ref, w_ref, y_ref):
    dinv = lax.rsqrt(d0_ref[...] + d1_ref[...] + 1.0)
    xw = jnp.dot(x_ref[...], w_ref[...], preferred_element_type=jnp.float32)
    y_ref[...] = xw * dinv


_y1_call = pl.pallas_call(
    _y1_body,
    grid=(NBLK,),
    in_specs=[
        pl.BlockSpec((BR, D), lambda i: (i, 0)),
        pl.BlockSpec((BR, 1), lambda i: (i, 0)),
        pl.BlockSpec((BR, 1), lambda i: (i, 0)),
        pl.BlockSpec((D, H), lambda i: (0, 0)),
    ],
    out_specs=pl.BlockSpec((BR, H), lambda i: (i, 0)),
    out_shape=jax.ShapeDtypeStruct((NP, H), jnp.float32),
)


def _layer_body(p0_ref, p1_ref, d0_ref, d1_ref, b_ref, w_ref, y_ref):
    dinv = lax.rsqrt(d0_ref[...] + d1_ref[...] + 1.0)
    t = (p0_ref[...] + p1_ref[...]) * dinv + b_ref[...]
    h = jnp.where(t > 0, t, 0.01 * t)
    y_ref[...] = jnp.dot(h, w_ref[...], preferred_element_type=jnp.float32) * dinv


_layer_call = pl.pallas_call(
    _layer_body,
    grid=(NBLK,),
    in_specs=[
        pl.BlockSpec((BR, H), lambda i: (i, 0)),
        pl.BlockSpec((BR, H), lambda i: (i, 0)),
        pl.BlockSpec((BR, 1), lambda i: (i, 0)),
        pl.BlockSpec((BR, 1), lambda i: (i, 0)),
        pl.BlockSpec((1, H), lambda i: (0, 0)),
        pl.BlockSpec((H, H), lambda i: (0, 0)),
    ],
    out_specs=pl.BlockSpec((BR, H), lambda i: (i, 0)),
    out_shape=jax.ShapeDtypeStruct((NP, H), jnp.float32),
)


def _final_body(p0_ref, p1_ref, d0_ref, d1_ref, b3_ref, batch_ref,
                wl_ref, bl_ref, wo_ref, bo_ref,
                logits_ref, probs_ref, feats_ref, embeds_ref, hout_ref,
                acc_ref, cnt_ref):
    i = pl.program_id(0)

    @pl.when(i == 0)
    def _():
        acc_ref[...] = jnp.zeros_like(acc_ref)
        cnt_ref[...] = jnp.zeros_like(cnt_ref)

    dinv = lax.rsqrt(d0_ref[...] + d1_ref[...] + 1.0)
    h3 = (p0_ref[...] + p1_ref[...]) * dinv + b3_ref[...]
    ids = batch_ref[0, 0, :]
    seg = lax.broadcasted_iota(jnp.int32, (G, BR), 0)
    onehot = jnp.where(seg == ids[None, :], 1.0, 0.0)
    acc_ref[...] += jnp.dot(onehot, h3, preferred_element_type=jnp.float32,
                            precision=lax.Precision.HIGHEST)
    cnt_ref[...] += jnp.sum(onehot, axis=1, keepdims=True)

    @pl.when(i == NBLK - 1)
    def _():
        summ = acc_ref[...]
        mean = summ / jnp.maximum(cnt_ref[...], 1.0)
        feats = jnp.concatenate([summ, mean], axis=1)
        embeds = jnp.dot(feats, wl_ref[...],
                         preferred_element_type=jnp.float32) + bl_ref[...]
        hout = jnp.maximum(embeds, 0.0)
        logits = jnp.dot(hout, wo_ref[...],
                         preferred_element_type=jnp.float32) + bo_ref[...]
        m = jnp.max(logits, axis=1, keepdims=True)
        ex = jnp.exp(logits - m)
        feats_ref[...] = feats
        embeds_ref[...] = embeds
        hout_ref[...] = hout
        logits_ref[...] = logits
        probs_ref[...] = ex / jnp.sum(ex, axis=1, keepdims=True)


_final_call = pl.pallas_call(
    _final_body,
    grid=(NBLK,),
    in_specs=[
        pl.BlockSpec((BR, H), lambda i: (i, 0)),
        pl.BlockSpec((BR, H), lambda i: (i, 0)),
        pl.BlockSpec((BR, 1), lambda i: (i, 0)),
        pl.BlockSpec((BR, 1), lambda i: (i, 0)),
        pl.BlockSpec((1, H), lambda i: (0, 0)),
        pl.BlockSpec((1, 1, BR), lambda i: (i, 0, 0)),
        pl.BlockSpec((2 * H, H), lambda i: (0, 0)),
        pl.BlockSpec((1, H), lambda i: (0, 0)),
        pl.BlockSpec((H, C), lambda i: (0, 0)),
        pl.BlockSpec((1, C), lambda i: (0, 0)),
    ],
    out_specs=[
        pl.BlockSpec((G, C), lambda i: (0, 0)),
        pl.BlockSpec((G, C), lambda i: (0, 0)),
        pl.BlockSpec((G, 2 * H), lambda i: (0, 0)),
        pl.BlockSpec((G, H), lambda i: (0, 0)),
        pl.BlockSpec((G, H), lambda i: (0, 0)),
    ],
    out_shape=[
        jax.ShapeDtypeStruct((G, C), jnp.float32),
        jax.ShapeDtypeStruct((G, C), jnp.float32),
        jax.ShapeDtypeStruct((G, 2 * H), jnp.float32),
        jax.ShapeDtypeStruct((G, H), jnp.float32),
        jax.ShapeDtypeStruct((G, H), jnp.float32),
    ],
    scratch_shapes=[
        pltpu.VMEM((G, H), jnp.float32),
        pltpu.VMEM((G, 1), jnp.float32),
    ],
)


def kernel(x, edge_index, batch, W1, b1, W2, b2, W3, b3, Wlin, blin, Wout, bout):
    pad_idx = N + (jnp.arange(EP - E, dtype=jnp.int32) % (NP - N))
    src4 = jnp.concatenate(
        [edge_index[0].astype(jnp.int32), pad_idx]
    ).reshape(NW, NCHUNK // DSTG, DSTG, EC)
    dst4 = jnp.concatenate(
        [edge_index[1].astype(jnp.int32), pad_idx]
    ).reshape(NW, NCHUNK // DSTG, DSTG, EC)
    batch3 = jnp.concatenate(
        [batch.astype(jnp.int32), jnp.full((NP - N,), G, jnp.int32)]
    ).reshape(NBLK, 1, BR)
    xp = jnp.pad(x, ((0, NP - N), (0, 0)))
    zeros_nd = jnp.zeros((NP, D), jnp.float32)

    deg_parts = _deg_call()(dst4)
    d0 = deg_parts[0].reshape(NP, 1)
    d1 = deg_parts[1].reshape(NP, 1)

    y1 = _y1_call(xp, d0, d1, W1)
    p1a, p1b = _edge_call()(y1, zeros_nd, src4, dst4)
    y2 = _layer_call(p1a, p1b, d0, d1, b1.reshape(1, H), W2)
    p2a, p2b = _edge_call()(y2, zeros_nd, src4, dst4)
    y3 = _layer_call(p2a, p2b, d0, d1, b2.reshape(1, H), W3)
    p3a, p3b = _edge_call()(y3, zeros_nd, src4, dst4)

    logits, probs, feats, embeds, hout = _final_call(
        p3a, p3b, d0, d1, b3.reshape(1, H), batch3,
        Wlin, blin.reshape(1, H), Wout, bout.reshape(1, C))
    return (logits, probs, feats, embeds, hout)
